# Initial kernel scaffold; baseline (speedup 1.0000x reference)
import jax
import jax.numpy as jnp
from jax.experimental import pallas as pl


def kernel(edge_index, x):
    raise NotImplementedError("write your pallas kernel here")


# speedup vs baseline: 9.3361x; 9.3361x over previous
import functools

import jax
import jax.numpy as jnp
from jax import lax
from jax.experimental import pallas as pl
from jax.experimental.pallas import tpu as pltpu
from jax.experimental.pallas import tpu_sc as plsc

N_NODES = 10000
N_EDGES = 320000
NUM_CORES = 2
NUM_SUBCORES = 16
NUM_TILES = NUM_CORES * NUM_SUBCORES
E_PER_TILE = N_EDGES // NUM_TILES
N_PAD = 10240
SLICE = N_PAD // NUM_SUBCORES
LANES = 16


def _sc_body(src_hbm, out_hbm, idx_v, ones_v, init_v, acc_s):
    c = lax.axis_index("c")
    s = lax.axis_index("s")
    wid = c * NUM_SUBCORES + s

    one = jnp.full((LANES,), 1.0, dtype=jnp.float32)
    zero = jnp.zeros((LANES,), dtype=jnp.float32)

    def fill_ones(i, _):
        ones_v[pl.ds(i * LANES, LANES)] = one
        return 0

    lax.fori_loop(0, E_PER_TILE // LANES, fill_ones, 0)

    @pl.when(s == 0)
    def _():
        def fill_zero(i, _):
            init_v[pl.ds(i * LANES, LANES)] = zero
            return 0

        lax.fori_loop(0, N_PAD // LANES, fill_zero, 0)
        pltpu.sync_copy(init_v, acc_s)

    pltpu.sync_copy(src_hbm.at[pl.ds(wid * E_PER_TILE, E_PER_TILE)], idx_v)

    plsc.subcore_barrier()

    pltpu.sync_copy(ones_v, acc_s.at[idx_v], add=True)

    plsc.subcore_barrier()

    pltpu.sync_copy(acc_s.at[pl.ds(s * SLICE, SLICE)],
                    out_hbm.at[c, pl.ds(s * SLICE, SLICE)])


@jax.jit
def _degree_partials(src):
    mesh = plsc.VectorSubcoreMesh(
        core_axis_name="c", subcore_axis_name="s",
        num_cores=NUM_CORES, num_subcores=NUM_SUBCORES)
    return pl.kernel(
        _sc_body,
        out_type=jax.ShapeDtypeStruct((NUM_CORES, N_PAD), jnp.float32),
        mesh=mesh,
        scratch_types=[
            pltpu.VMEM((E_PER_TILE,), jnp.int32),
            pltpu.VMEM((E_PER_TILE,), jnp.float32),
            pltpu.VMEM((N_PAD,), jnp.float32),
            pltpu.VMEM_SHARED((N_PAD,), jnp.float32),
        ],
    )(src)


def _combine_body(p_ref, o_ref):
    o_ref[...] = p_ref[0] + p_ref[1] + 1.0


@jax.jit
def _combine(partials):
    p = partials.reshape(NUM_CORES, N_PAD // 128, 128)
    out = pl.pallas_call(
        _combine_body,
        out_shape=jax.ShapeDtypeStruct((N_PAD // 128, 128), jnp.float32),
    )(p)
    return out.reshape(N_PAD)[:N_NODES]


def kernel(edge_index, x):
    src = edge_index[0]
    partials = _degree_partials(src)
    return _combine(partials)


# speedup vs baseline: 9.5393x; 1.0218x over previous
import jax
import jax.numpy as jnp
from jax import lax
from jax.experimental import pallas as pl
from jax.experimental.pallas import tpu as pltpu
from jax.experimental.pallas import tpu_sc as plsc

N_NODES = 10000
N_EDGES = 320000
NUM_CORES = 2
NUM_SUBCORES = 16
NUM_TILES = NUM_CORES * NUM_SUBCORES
E_PER_TILE = N_EDGES // NUM_TILES
N_PAD = 10240
LANES = 16


def _sc_body(src_hbm, out_hbm, idx_v, hist_v):
    c = lax.axis_index("c")
    s = lax.axis_index("s")
    wid = c * NUM_SUBCORES + s

    pltpu.sync_copy(src_hbm.at[pl.ds(wid * E_PER_TILE, E_PER_TILE)], idx_v)

    zero = jnp.zeros((LANES,), dtype=jnp.float32)

    def fill_zero(i, _):
        hist_v[pl.ds(i * LANES, LANES)] = zero
        return 0

    lax.fori_loop(0, N_PAD // LANES, fill_zero, 0)

    one = jnp.full((LANES,), 1.0, dtype=jnp.float32)

    def accum(i, _):
        idx = idx_v[pl.ds(i * LANES, LANES)]
        plsc.addupdate_scatter(hist_v, [idx], one)
        return 0

    lax.fori_loop(0, E_PER_TILE // LANES, accum, 0)

    pltpu.sync_copy(hist_v, out_hbm.at[wid])


@jax.jit
def _degree_partials(src):
    mesh = plsc.VectorSubcoreMesh(
        core_axis_name="c", subcore_axis_name="s",
        num_cores=NUM_CORES, num_subcores=NUM_SUBCORES)
    return pl.kernel(
        _sc_body,
        out_type=jax.ShapeDtypeStruct((NUM_TILES, N_PAD), jnp.float32),
        mesh=mesh,
        scratch_types=[
            pltpu.VMEM((E_PER_TILE,), jnp.int32),
            pltpu.VMEM((N_PAD,), jnp.float32),
        ],
        compiler_params=pltpu.CompilerParams(needs_layout_passes=False),
    )(src)


def _combine_body(p_ref, o_ref):
    o_ref[...] = jnp.sum(p_ref[...], axis=0) + 1.0


@jax.jit
def _combine(partials):
    p = partials.reshape(NUM_TILES, N_PAD // 128, 128)
    out = pl.pallas_call(
        _combine_body,
        out_shape=jax.ShapeDtypeStruct((N_PAD // 128, 128), jnp.float32),
    )(p)
    return out.reshape(N_PAD)[:N_NODES]


def kernel(edge_index, x):
    partials = _degree_partials(edge_index[0])
    return _combine(partials)


# speedup vs baseline: 10.1410x; 1.0631x over previous
import jax
import jax.numpy as jnp
from jax import lax
from jax.experimental import pallas as pl
from jax.experimental.pallas import tpu as pltpu
from jax.experimental.pallas import tpu_sc as plsc

N_NODES = 10000
N_EDGES = 320000
NUM_CORES = 2
NUM_SUBCORES = 16
NUM_TILES = NUM_CORES * NUM_SUBCORES
E_PER_TILE = N_EDGES // NUM_TILES
N_PAD = 10240
LANES = 16
UNROLL = 5


def _sc_body(src_hbm, out_hbm, idx_v, hist_v, sem):
    c = lax.axis_index("c")
    s = lax.axis_index("s")
    wid = c * NUM_SUBCORES + s

    copy = pltpu.async_copy(
        src_hbm.at[pl.ds(wid * E_PER_TILE, E_PER_TILE)], idx_v, sem)

    zero = jnp.zeros((LANES,), dtype=jnp.float32)

    def fill_zero(i, _):
        for u in range(8):
            hist_v[pl.ds((i * 8 + u) * LANES, LANES)] = zero
        return 0

    lax.fori_loop(0, N_PAD // (8 * LANES), fill_zero, 0)

    copy.wait()

    one = jnp.full((LANES,), 1.0, dtype=jnp.float32)

    def accum(i, _):
        for u in range(UNROLL):
            idx = idx_v[pl.ds((i * UNROLL + u) * LANES, LANES)]
            plsc.addupdate_scatter(hist_v, [idx], one)
        return 0

    lax.fori_loop(0, E_PER_TILE // (UNROLL * LANES), accum, 0)

    pltpu.sync_copy(hist_v, out_hbm.at[wid])


@jax.jit
def _degree_partials(src):
    mesh = plsc.VectorSubcoreMesh(
        core_axis_name="c", subcore_axis_name="s",
        num_cores=NUM_CORES, num_subcores=NUM_SUBCORES)
    return pl.kernel(
        _sc_body,
        out_type=jax.ShapeDtypeStruct((NUM_TILES, N_PAD), jnp.float32),
        mesh=mesh,
        scratch_types=[
            pltpu.VMEM((E_PER_TILE,), jnp.int32),
            pltpu.VMEM((N_PAD,), jnp.float32),
            pltpu.SemaphoreType.DMA,
        ],
        compiler_params=pltpu.CompilerParams(needs_layout_passes=False),
    )(src)


def _combine_body(p_ref, o_ref):
    o_ref[...] = jnp.sum(p_ref[...], axis=0) + 1.0


@jax.jit
def _combine(partials):
    p = partials.reshape(NUM_TILES, N_PAD // 128, 128)
    out = pl.pallas_call(
        _combine_body,
        out_shape=jax.ShapeDtypeStruct((N_PAD // 128, 128), jnp.float32),
    )(p)
    return out.reshape(N_PAD)[:N_NODES]


def kernel(edge_index, x):
    src = edge_index.reshape(-1)[:N_EDGES]
    partials = _degree_partials(src)
    return _combine(partials)


# speedup vs baseline: 13.4402x; 1.3253x over previous
import jax
import jax.numpy as jnp
from jax import lax
from jax.experimental import pallas as pl
from jax.experimental.pallas import tpu as pltpu
from jax.experimental.pallas import tpu_sc as plsc

N_NODES = 10000
N_EDGES = 320000
NUM_CORES = 2
NUM_SUBCORES = 16
NUM_TILES = NUM_CORES * NUM_SUBCORES
E_PER_TILE = N_EDGES // NUM_TILES
N_PAD = 10240
LANES = 16
UNROLL = 5


def _sc_body(src_hbm, out_hbm, idx_v, hist_v, sem):
    c = lax.axis_index("c")
    s = lax.axis_index("s")
    wid = c * NUM_SUBCORES + s

    copy = pltpu.async_copy(
        src_hbm.at[pl.ds(wid * E_PER_TILE, E_PER_TILE)], idx_v, sem)

    zero = jnp.zeros((LANES,), dtype=jnp.float32)

    def fill_zero(i, _):
        for u in range(8):
            hist_v[pl.ds((i * 8 + u) * LANES, LANES)] = zero
        return 0

    lax.fori_loop(0, N_PAD // (8 * LANES), fill_zero, 0)

    copy.wait()

    one = jnp.full((LANES,), 1.0, dtype=jnp.float32)

    def accum(i, _):
        for u in range(UNROLL):
            idx = idx_v[pl.ds((i * UNROLL + u) * LANES, LANES)]
            plsc.addupdate_scatter(hist_v, [idx], one)
        return 0

    lax.fori_loop(0, E_PER_TILE // (UNROLL * LANES), accum, 0)

    pltpu.sync_copy(hist_v, out_hbm.at[wid])


@jax.jit
def _degree_partials(src):
    mesh = plsc.VectorSubcoreMesh(
        core_axis_name="c", subcore_axis_name="s",
        num_cores=NUM_CORES, num_subcores=NUM_SUBCORES)
    return pl.kernel(
        _sc_body,
        out_type=jax.ShapeDtypeStruct((NUM_TILES, N_PAD), jnp.float32),
        mesh=mesh,
        scratch_types=[
            pltpu.VMEM((E_PER_TILE,), jnp.int32),
            pltpu.VMEM((N_PAD,), jnp.float32),
            pltpu.SemaphoreType.DMA,
        ],
        compiler_params=pltpu.CompilerParams(needs_layout_passes=False),
    )(src)


def _combine_body(p_ref, o_ref):
    o_ref[...] = jnp.sum(p_ref[...], axis=0) + 1.0


@jax.jit
def _combine(partials):
    p = partials.reshape(NUM_TILES, N_PAD // 128, 128)
    out = pl.pallas_call(
        _combine_body,
        out_shape=jax.ShapeDtypeStruct((N_PAD // 128, 128), jnp.float32),
    )(p)
    return out.reshape(N_PAD)[:N_NODES]


def kernel(edge_index, x):
    src = edge_index.reshape(-1)
    partials = _degree_partials(src)
    return _combine(partials)


# speedup vs baseline: 13.7171x; 1.0206x over previous
import jax
import jax.numpy as jnp
from jax import lax
from jax.experimental import pallas as pl
from jax.experimental.pallas import tpu as pltpu
from jax.experimental.pallas import tpu_sc as plsc

N_NODES = 10000
N_EDGES = 320000
NUM_SUBCORES = 16
E_PER_TILE = N_EDGES // NUM_SUBCORES
N_PAD = 10240
BINS_PER_TILE = N_PAD // NUM_SUBCORES
LANES = 16
UNROLL = 5


def _sc_body(src_hbm, out_hbm, idx_v, hist_v, red_v, col_v, sum_v, sem):
    s = lax.axis_index("s")

    copy = pltpu.async_copy(
        src_hbm.at[pl.ds(s * E_PER_TILE, E_PER_TILE)], idx_v, sem)

    zero = jnp.zeros((LANES,), dtype=jnp.float32)

    def fill_zero(i, _):
        for u in range(8):
            hist_v[pl.ds((i * 8 + u) * LANES, LANES)] = zero
        return 0

    lax.fori_loop(0, N_PAD // (8 * LANES), fill_zero, 0)

    copy.wait()

    one = jnp.full((LANES,), 1.0, dtype=jnp.float32)

    def accum(i, _):
        for u in range(UNROLL):
            idx = idx_v[pl.ds((i * UNROLL + u) * LANES, LANES)]
            plsc.addupdate_scatter(hist_v, [idx], one)
        return 0

    lax.fori_loop(0, E_PER_TILE // (UNROLL * LANES), accum, 0)

    pltpu.sync_copy(hist_v, red_v.at[s])
    plsc.subcore_barrier()

    pltpu.sync_copy(red_v.at[:, pl.ds(s * BINS_PER_TILE, BINS_PER_TILE)],
                    col_v)

    def reduce_vec(j, _):
        acc = one
        for r in range(NUM_SUBCORES):
            acc = acc + col_v[r, pl.ds(j * LANES, LANES)]
        sum_v[pl.ds(j * LANES, LANES)] = acc
        return 0

    lax.fori_loop(0, BINS_PER_TILE // LANES, reduce_vec, 0)

    pltpu.sync_copy(sum_v, out_hbm.at[pl.ds(s * BINS_PER_TILE, BINS_PER_TILE)])


@jax.jit
def _degree(src):
    mesh = plsc.VectorSubcoreMesh(
        core_axis_name="c", subcore_axis_name="s",
        num_cores=1, num_subcores=NUM_SUBCORES)
    return pl.kernel(
        _sc_body,
        out_type=jax.ShapeDtypeStruct((N_PAD,), jnp.float32),
        mesh=mesh,
        scratch_types=[
            pltpu.VMEM((E_PER_TILE,), jnp.int32),
            pltpu.VMEM((N_PAD,), jnp.float32),
            pltpu.VMEM_SHARED((NUM_SUBCORES, N_PAD), jnp.float32),
            pltpu.VMEM((NUM_SUBCORES, BINS_PER_TILE), jnp.float32),
            pltpu.VMEM((BINS_PER_TILE,), jnp.float32),
            pltpu.SemaphoreType.DMA,
        ],
        compiler_params=pltpu.CompilerParams(needs_layout_passes=False),
    )(src)


def kernel(edge_index, x):
    src = edge_index.reshape(-1)
    return _degree(src)[:N_NODES]
